# baseline (speedup 1.0000x reference)
import functools

import jax
import jax.numpy as jnp
from jax import lax
from jax.experimental import pallas as pl
from jax.experimental.pallas import tpu as pltpu
from jax.experimental.pallas import tpu_sc as plsc

NC = 2
NS = 16
L = 16
NW = NC * NS

IDXC = 128
FUNROLL = 16


def _rsqrt(x):
    xi = plsc.bitcast(x, jnp.int32)
    r = plsc.bitcast(jnp.full((L,), 0x5F3759DF, jnp.int32) - (xi >> 1),
                     jnp.float32)
    for _ in range(3):
        r = r * (1.5 - 0.5 * x * r * r)
    return r


@functools.lru_cache(maxsize=None)
def _build(B, D, C):
    b_per_w = B // NW
    h_per_s = B // NS
    hch = h_per_s // IDXC
    cpad = ((C + 8 * NS - 1) // (8 * NS)) * (8 * NS)
    z_per_s = cpad // NS
    n_chunks = b_per_w // IDXC
    blocks = IDXC // L

    mesh = plsc.VectorSubcoreMesh(core_axis_name="c", subcore_axis_name="s")

    @functools.partial(
        pl.kernel,
        mesh=mesh,
        out_type=jax.ShapeDtypeStruct((NC, L), jnp.float32),
        compiler_params=pltpu.CompilerParams(needs_layout_passes=False),
        scratch_types=[
            pltpu.VMEM((z_per_s,), jnp.float32),
            pltpu.VMEM((IDXC,), jnp.float32),
            [pltpu.VMEM((IDXC,), jnp.int32) for _ in range(8)],
            [pltpu.VMEM((IDXC,), jnp.int32) for _ in range(2)],
            [pltpu.VMEM((IDXC,), jnp.float32) for _ in range(4)],
            [pltpu.VMEM((IDXC, D), jnp.float32) for _ in range(2)],
            [pltpu.VMEM((IDXC, D), jnp.float32) for _ in range(2)],
            pltpu.VMEM((B // NW,), jnp.float32),
            pltpu.VMEM((L,), jnp.float32),
            pltpu.VMEM((NS * L,), jnp.float32),
            pltpu.VMEM_SHARED((cpad,), jnp.float32),
            pltpu.VMEM_SHARED((NS * L,), jnp.float32),
            pltpu.SemaphoreType.DMA,
            pltpu.SemaphoreType.DMA,
            [pltpu.SemaphoreType.DMA for _ in range(2)],
            [pltpu.SemaphoreType.DMA for _ in range(2)],
            [pltpu.SemaphoreType.DMA for _ in range(2)],
        ],
    )
    def sc_kernel(xs_hbm, ys_hbm, center_hbm, out_hbm,
                  zero_v, ones_v, hbuf, idx_b, cnt4, rows_b, xsc_b, dist_v,
                  lane_v, pread_v, count_sh, part_sh, semh, sema,
                  semr, semx, semc):
        cid = lax.axis_index("c")
        sid = lax.axis_index("s")
        wid = sid * NC + cid
        iota = lax.iota(jnp.int32, L)
        base = wid * b_per_w

        zv = jnp.zeros((L,), jnp.float32)
        ov = jnp.ones((L,), jnp.float32)

        def zfill(i, _):
            zero_v[pl.ds(i * L, L)] = zv
            return 0
        lax.fori_loop(0, z_per_s // L, zfill, 0)
        for i in range(IDXC // L):
            ones_v[pl.ds(i * L, L)] = ov

        pend = [None, None]
        for b in range(min(2, n_chunks)):
            off = base + b * IDXC
            pltpu.sync_copy(ys_hbm.at[pl.ds(off, IDXC)], idx_b[b])
            pend[b] = [
                pltpu.async_copy(center_hbm.at[idx_b[b]], rows_b[b], semr[b]),
                pltpu.async_copy(xs_hbm.at[pl.ds(off, IDXC)], xsc_b[b],
                                 semx[b]),
            ]

        pltpu.sync_copy(zero_v, count_sh.at[pl.ds(sid * z_per_s, z_per_s)])
        plsc.subcore_barrier()
        hbase = sid * h_per_s
        lcps = [pltpu.async_copy(ys_hbm.at[pl.ds(hbase + j * IDXC, IDXC)],
                                 hbuf[j], semh) for j in range(hch)]
        for c in lcps:
            c.wait()
        acps = [pltpu.async_copy(ones_v, count_sh.at[hbuf[j]], sema, add=True)
                for j in range(hch)]

        def issue(ci, buf):
            off = base + ci * IDXC
            pltpu.sync_copy(ys_hbm.at[pl.ds(off, IDXC)], idx_b[buf])
            return [
                pltpu.async_copy(center_hbm.at[idx_b[buf]], rows_b[buf],
                                 semr[buf]),
                pltpu.async_copy(xs_hbm.at[pl.ds(off, IDXC)], xsc_b[buf],
                                 semx[buf]),
            ]

        for ci in range(n_chunks):
            buf = ci % 2
            for c in pend[buf]:
                c.wait()
            rows_v = rows_b[buf]
            xsc_v = xsc_b[buf]

            def bbody(b, _):
                rowids = b * L + iota

                def fbody(f8, acc):
                    a = acc
                    for u in range(FUNROLL):
                        f = f8 * FUNROLL + u
                        colv0 = f + iota
                        colv = jnp.where(colv0 >= D, colv0 - D, colv0)
                        xv = plsc.load_gather(xsc_v, [rowids, colv])
                        cv = plsc.load_gather(rows_v, [rowids, colv])
                        d = xv - cv
                        a = a + d * d
                    return a

                acc = lax.fori_loop(0, D // FUNROLL, fbody,
                                    jnp.zeros((L,), jnp.float32))
                x = jnp.maximum(acc, 1e-30)
                dist_v[pl.ds(ci * IDXC + b * L, L)] = x * _rsqrt(x)
                return 0

            lax.fori_loop(0, blocks, bbody, 0)
            if ci + 2 < n_chunks:
                pend[buf] = issue(ci + 2, buf)

        for c in acps:
            c.wait()
        plsc.subcore_barrier()
        rcps = [pltpu.async_copy(ys_hbm.at[pl.ds(base + j * IDXC, IDXC)],
                                 hbuf[j], semh) for j in range(n_chunks)]
        for c in rcps:
            c.wait()
        ccps = [pltpu.async_copy(count_sh.at[hbuf[j]], cnt4[j], semc[0])
                for j in range(n_chunks)]
        for c in ccps:
            c.wait()

        loss = jnp.zeros((L,), jnp.float32)
        for j in range(n_chunks):
            def dbody(b, lacc):
                dv = dist_v[pl.ds(j * IDXC + b * L, L)]
                cv = cnt4[j][pl.ds(b * L, L)]
                return lacc + dv / cv
            loss = lax.fori_loop(0, blocks, dbody, loss)

        lane_v[...] = loss
        pltpu.sync_copy(lane_v, part_sh.at[pl.ds(sid * L, L)])
        plsc.subcore_barrier()

        @pl.when(sid == 0)
        def _():
            pltpu.sync_copy(part_sh, pread_v)
            tv = jnp.zeros((L,), jnp.float32)
            for i in range(NS):
                tv = tv + pread_v[pl.ds(i * L, L)]
            s = jnp.sum(tv)
            lane_v[...] = jnp.full((L,), s)
            pltpu.sync_copy(lane_v, out_hbm.at[cid])

    return sc_kernel


def kernel(xs, ys, center):
    B, D = xs.shape
    C = center.shape[0]
    out = _build(B, D, C)(xs, ys.astype(jnp.int32), center)
    return out[0, 0] + out[1, 0]


# speedup vs baseline: 1.0372x; 1.0372x over previous
import functools

import jax
import jax.numpy as jnp
from jax import lax
from jax.experimental import pallas as pl
from jax.experimental.pallas import tpu as pltpu
from jax.experimental.pallas import tpu_sc as plsc

NC = 2
NS = 16
L = 16
NW = NC * NS

IDXC = 128
FUNROLL = 16


def _rsqrt(x):
    xi = plsc.bitcast(x, jnp.int32)
    r = plsc.bitcast(jnp.full((L,), 0x5F3759DF, jnp.int32) - (xi >> 1),
                     jnp.float32)
    for _ in range(3):
        r = r * (1.5 - 0.5 * x * r * r)
    return r


@functools.lru_cache(maxsize=None)
def _build(B, D, C):
    b_per_w = B // NW
    h_per_s = B // NS
    hch = h_per_s // IDXC
    cpad = ((C + 8 * NS - 1) // (8 * NS)) * (8 * NS)
    z_per_s = cpad // NS
    n_chunks = b_per_w // IDXC
    blocks = IDXC // L

    mesh = plsc.VectorSubcoreMesh(core_axis_name="c", subcore_axis_name="s")

    @functools.partial(
        pl.kernel,
        mesh=mesh,
        out_type=jax.ShapeDtypeStruct((NC, L), jnp.float32),
        compiler_params=pltpu.CompilerParams(needs_layout_passes=False),
        scratch_types=[
            pltpu.VMEM((z_per_s,), jnp.float32),
            pltpu.VMEM((IDXC,), jnp.float32),
            [pltpu.VMEM((IDXC,), jnp.int32) for _ in range(8)],
            [pltpu.VMEM((IDXC,), jnp.int32) for _ in range(2)],
            [pltpu.VMEM((IDXC,), jnp.float32) for _ in range(4)],
            [pltpu.VMEM((IDXC, D), jnp.float32) for _ in range(2)],
            [pltpu.VMEM((IDXC, D), jnp.float32) for _ in range(2)],
            pltpu.VMEM((B // NW,), jnp.float32),
            pltpu.VMEM((L,), jnp.float32),
            pltpu.VMEM((NS * L,), jnp.float32),
            pltpu.VMEM_SHARED((cpad,), jnp.float32),
            pltpu.VMEM_SHARED((NS * L,), jnp.float32),
            pltpu.SemaphoreType.DMA,
            pltpu.SemaphoreType.DMA,
            [pltpu.SemaphoreType.DMA for _ in range(2)],
            [pltpu.SemaphoreType.DMA for _ in range(2)],
            [pltpu.SemaphoreType.DMA for _ in range(2)],
        ],
    )
    def sc_kernel(xs_hbm, ys_hbm, center_hbm, out_hbm,
                  zero_v, ones_v, hbuf, idx_b, cnt4, rows_b, xsc_b, dist_v,
                  lane_v, pread_v, count_sh, part_sh, semh, sema,
                  semr, semx, semc):
        cid = lax.axis_index("c")
        sid = lax.axis_index("s")
        wid = sid * NC + cid
        iota = lax.iota(jnp.int32, L)
        base = wid * b_per_w

        zv = jnp.zeros((L,), jnp.float32)
        ov = jnp.ones((L,), jnp.float32)

        def zfill(i, _):
            zero_v[pl.ds(i * L, L)] = zv
            return 0
        lax.fori_loop(0, z_per_s // L, zfill, 0)
        for i in range(IDXC // L):
            ones_v[pl.ds(i * L, L)] = ov

        pltpu.sync_copy(ys_hbm.at[pl.ds(base, IDXC)], idx_b[0])
        pend = [None, None]
        pend[0] = [
            pltpu.async_copy(center_hbm.at[idx_b[0]], rows_b[0], semr[0]),
            pltpu.async_copy(xs_hbm.at[pl.ds(base, IDXC)], xsc_b[0], semx[0]),
        ]

        pltpu.sync_copy(zero_v, count_sh.at[pl.ds(sid * z_per_s, z_per_s)])
        plsc.subcore_barrier()
        hbase = sid * h_per_s
        lcps = [pltpu.async_copy(ys_hbm.at[pl.ds(hbase + j * IDXC, IDXC)],
                                 hbuf[j], semh) for j in range(hch)]
        for c in lcps:
            c.wait()
        acps = [pltpu.async_copy(ones_v, count_sh.at[hbuf[j]], sema, add=True)
                for j in range(hch)]
        for c in acps:
            c.wait()
        plsc.subcore_barrier()

        pend[0].append(
            pltpu.async_copy(count_sh.at[idx_b[0]], cnt4[0], semc[0]))

        def issue(ci, buf):
            off = base + ci * IDXC
            pltpu.sync_copy(ys_hbm.at[pl.ds(off, IDXC)], idx_b[buf])
            return [
                pltpu.async_copy(center_hbm.at[idx_b[buf]], rows_b[buf],
                                 semr[buf]),
                pltpu.async_copy(xs_hbm.at[pl.ds(off, IDXC)], xsc_b[buf],
                                 semx[buf]),
                pltpu.async_copy(count_sh.at[idx_b[buf]], cnt4[buf],
                                 semc[buf]),
            ]

        loss = jnp.zeros((L,), jnp.float32)
        for ci in range(n_chunks):
            buf = ci % 2
            if ci + 1 < n_chunks:
                pend[1 - buf] = issue(ci + 1, 1 - buf)
            for c in pend[buf]:
                c.wait()
            rows_v = rows_b[buf]
            xsc_v = xsc_b[buf]
            cnt_v = cnt4[buf]

            def bbody(b, lacc):
                rowids = b * L + iota

                def fbody(f8, acc):
                    a = acc
                    for u in range(FUNROLL):
                        f = f8 * FUNROLL + u
                        colv0 = f + iota
                        colv = jnp.where(colv0 >= D, colv0 - D, colv0)
                        xv = plsc.load_gather(xsc_v, [rowids, colv])
                        cv = plsc.load_gather(rows_v, [rowids, colv])
                        d = xv - cv
                        a = a + d * d
                    return a

                acc = lax.fori_loop(0, D // FUNROLL, fbody,
                                    jnp.zeros((L,), jnp.float32))
                x = jnp.maximum(acc, 1e-30)
                dist = x * _rsqrt(x)
                cnt16 = cnt_v[pl.ds(b * L, L)]
                return lacc + dist / cnt16

            loss = lax.fori_loop(0, blocks, bbody, loss)

        lane_v[...] = loss
        pltpu.sync_copy(lane_v, part_sh.at[pl.ds(sid * L, L)])
        plsc.subcore_barrier()

        @pl.when(sid == 0)
        def _():
            pltpu.sync_copy(part_sh, pread_v)
            tv = jnp.zeros((L,), jnp.float32)
            for i in range(NS):
                tv = tv + pread_v[pl.ds(i * L, L)]
            s = jnp.sum(tv)
            lane_v[...] = jnp.full((L,), s)
            pltpu.sync_copy(lane_v, out_hbm.at[cid])

    return sc_kernel


def kernel(xs, ys, center):
    B, D = xs.shape
    C = center.shape[0]
    out = _build(B, D, C)(xs, ys.astype(jnp.int32), center)
    return out[0, 0] + out[1, 0]


# speedup vs baseline: 1.1187x; 1.0786x over previous
import functools

import jax
import jax.numpy as jnp
from jax import lax
from jax.experimental import pallas as pl
from jax.experimental.pallas import tpu as pltpu
from jax.experimental.pallas import tpu_sc as plsc

NC = 2
NS = 16
L = 16
NW = NC * NS

IDXC = 128
FUNROLL = 16


def _rsqrt(x):
    xi = plsc.bitcast(x, jnp.int32)
    r = plsc.bitcast(jnp.full((L,), 0x5F3759DF, jnp.int32) - (xi >> 1),
                     jnp.float32)
    for _ in range(3):
        r = r * (1.5 - 0.5 * x * r * r)
    return r


@functools.lru_cache(maxsize=None)
def _build(B, D, C):
    b_per_w = B // NW
    h_per_s = B // NS
    hch = h_per_s // IDXC
    cpad = ((C + 8 * NS - 1) // (8 * NS)) * (8 * NS)
    z_per_s = cpad // NS
    n_chunks = b_per_w // IDXC
    blocks = IDXC // L

    mesh = plsc.VectorSubcoreMesh(core_axis_name="c", subcore_axis_name="s")

    @functools.partial(
        pl.kernel,
        mesh=mesh,
        out_type=jax.ShapeDtypeStruct((NC, L), jnp.float32),
        compiler_params=pltpu.CompilerParams(needs_layout_passes=False),
        scratch_types=[
            pltpu.VMEM((z_per_s,), jnp.float32),
            pltpu.VMEM((IDXC,), jnp.float32),
            [pltpu.VMEM((IDXC,), jnp.int32) for _ in range(8)],
            [pltpu.VMEM((IDXC,), jnp.int32) for _ in range(2)],
            [pltpu.VMEM((IDXC,), jnp.float32) for _ in range(4)],
            [pltpu.VMEM((IDXC, D), jnp.float32) for _ in range(2)],
            [pltpu.VMEM((IDXC, D), jnp.float32) for _ in range(2)],
            pltpu.VMEM((B // NW,), jnp.float32),
            pltpu.VMEM((L,), jnp.float32),
            pltpu.VMEM((NS * L,), jnp.float32),
            pltpu.VMEM_SHARED((cpad,), jnp.float32),
            pltpu.VMEM_SHARED((NS * L,), jnp.float32),
            pltpu.SemaphoreType.DMA,
            pltpu.SemaphoreType.DMA,
            [pltpu.SemaphoreType.DMA for _ in range(2)],
            [pltpu.SemaphoreType.DMA for _ in range(2)],
            [pltpu.SemaphoreType.DMA for _ in range(2)],
        ],
    )
    def sc_kernel(xs_hbm, ys_hbm, center_hbm, out_hbm,
                  zero_v, ones_v, hbuf, idx_b, cnt4, rows_b, xsc_b, dist_v,
                  lane_v, pread_v, count_sh, part_sh, semh, sema,
                  semr, semx, semc):
        cid = lax.axis_index("c")
        sid = lax.axis_index("s")
        wid = sid * NC + cid
        iota = lax.iota(jnp.int32, L)
        base = wid * b_per_w

        zv = jnp.zeros((L,), jnp.float32)
        ov = jnp.ones((L,), jnp.float32)

        def zfill(i, _):
            zero_v[pl.ds(i * L, L)] = zv
            return 0
        lax.fori_loop(0, z_per_s // L, zfill, 0)
        for i in range(IDXC // L):
            ones_v[pl.ds(i * L, L)] = ov

        pltpu.sync_copy(ys_hbm.at[pl.ds(base, IDXC)], idx_b[0])
        pend = [None, None]
        pend[0] = [
            pltpu.async_copy(center_hbm.at[idx_b[0]], rows_b[0], semr[0]),
            pltpu.async_copy(xs_hbm.at[pl.ds(base, IDXC)], xsc_b[0], semx[0]),
        ]

        pltpu.sync_copy(zero_v, count_sh.at[pl.ds(sid * z_per_s, z_per_s)])
        plsc.subcore_barrier()
        hbase = sid * h_per_s
        lcps = [pltpu.async_copy(ys_hbm.at[pl.ds(hbase + j * IDXC, IDXC)],
                                 hbuf[j], semh) for j in range(hch)]
        for c in lcps:
            c.wait()
        acps = [pltpu.async_copy(ones_v, count_sh.at[hbuf[j]], sema, add=True)
                for j in range(hch)]
        for c in acps:
            c.wait()
        plsc.subcore_barrier()

        cpend = [None, None]
        cpend[0] = pltpu.async_copy(count_sh.at[idx_b[0]], cnt4[0], semc[0])

        def issue(ci, buf):
            off = base + ci * IDXC
            pltpu.sync_copy(ys_hbm.at[pl.ds(off, IDXC)], idx_b[buf])
            cpend[buf] = pltpu.async_copy(count_sh.at[idx_b[buf]], cnt4[buf],
                                          semc[buf])
            return [
                pltpu.async_copy(center_hbm.at[idx_b[buf]], rows_b[buf],
                                 semr[buf]),
                pltpu.async_copy(xs_hbm.at[pl.ds(off, IDXC)], xsc_b[buf],
                                 semx[buf]),
            ]

        loss = jnp.zeros((L,), jnp.float32)
        for ci in range(n_chunks):
            buf = ci % 2
            if ci + 1 < n_chunks:
                pend[1 - buf] = issue(ci + 1, 1 - buf)
            for c in pend[buf]:
                c.wait()
            rows_v = rows_b[buf]
            xsc_v = xsc_b[buf]
            cnt_v = cnt4[buf]

            def bbody(b, _):
                rowids = b * L + iota

                def fbody(f8, acc):
                    a = acc
                    for u in range(FUNROLL):
                        f = f8 * FUNROLL + u
                        colv0 = f + iota
                        if D & (D - 1) == 0:
                            colv = colv0 & (D - 1)
                        else:
                            colv = jnp.where(colv0 >= D, colv0 - D, colv0)
                        xv = plsc.load_gather(xsc_v, [rowids, colv])
                        cv = plsc.load_gather(rows_v, [rowids, colv])
                        d = xv - cv
                        a = a + d * d
                    return a

                acc = lax.fori_loop(0, D // FUNROLL, fbody,
                                    jnp.zeros((L,), jnp.float32))
                x = jnp.maximum(acc, 1e-30)
                dist_v[pl.ds(b * L, L)] = x * _rsqrt(x)
                return 0

            lax.fori_loop(0, blocks, bbody, 0)
            cpend[buf].wait()

            def dbody(b, lacc):
                dv = dist_v[pl.ds(b * L, L)]
                cv = cnt_v[pl.ds(b * L, L)]
                return lacc + dv / cv
            loss = lax.fori_loop(0, blocks, dbody, loss)

        lane_v[...] = loss
        pltpu.sync_copy(lane_v, part_sh.at[pl.ds(sid * L, L)])
        plsc.subcore_barrier()

        @pl.when(sid == 0)
        def _():
            pltpu.sync_copy(part_sh, pread_v)
            tv = jnp.zeros((L,), jnp.float32)
            for i in range(NS):
                tv = tv + pread_v[pl.ds(i * L, L)]
            s = jnp.sum(tv)
            lane_v[...] = jnp.full((L,), s)
            pltpu.sync_copy(lane_v, out_hbm.at[cid])

    return sc_kernel


def kernel(xs, ys, center):
    B, D = xs.shape
    C = center.shape[0]
    out = _build(B, D, C)(xs, ys.astype(jnp.int32), center)
    return out[0, 0] + out[1, 0]
